# Initial kernel scaffold; baseline (speedup 1.0000x reference)
import jax
import jax.numpy as jnp
from jax.experimental import pallas as pl


def kernel(x, edge_index, W1, a_src1, a_dst1, b1, W2, a_src2, a_dst2, b2):
    raise NotImplementedError("write your pallas kernel here")


# speedup vs baseline: 21.1468x; 21.1468x over previous
import jax
import jax.numpy as jnp
from jax import lax
from jax.experimental import pallas as pl
from jax.experimental.pallas import tpu as pltpu
from jax.experimental.pallas import tpu_sc as plsc

N = 10000
NA = 10112
NP = 10240
E = 160000
IN_CH = 128
HID = 400
OUT_CH = 4
NC = 2
NS = 16
NW = NC * NS
EPT = E // NW
RPT = NP // NS
G = 128
NCH = -(-EPT // G)
TAIL_CH = EPT // G
TAIL_OFF = EPT - TAIL_CH * G

_MESH = plsc.VectorSubcoreMesh(core_axis_name="c", subcore_axis_name="s")
_SC_PARAMS = pltpu.CompilerParams(needs_layout_passes=False)


def _stage_common(als, ald, dst3, wid, als_v, ald_v, dst2):
  pltpu.sync_copy(als, als_v.at[pl.ds(0, N)])
  pltpu.sync_copy(ald, ald_v.at[pl.ds(0, N)])
  pltpu.sync_copy(dst3.at[wid], dst2)


def _compute_ee(src3, wid, als_v, ald_v, srcb, dst2, ee2):
  def _chunk(j, _):
    pltpu.sync_copy(src3.at[wid, j], srcb.at[0])
    for m in range(G // 16):
      sl = pl.ds(m * 16, 16)
      s16 = srcb[0, sl]
      d16 = dst2[j, sl]
      a = plsc.load_gather(als_v, [s16])
      b = plsc.load_gather(ald_v, [d16])
      e = a + b
      e = jnp.where(e > 0, e, 0.2 * e)
      ee2[j, sl] = jnp.exp(e)
    return 0
  lax.fori_loop(0, NCH, _chunk, 0)

  lanes = lax.iota(jnp.int32, 16)
  zeros16 = jnp.zeros((16,), jnp.float32)
  v = ee2[TAIL_CH, pl.ds(0, 16)]
  ee2[TAIL_CH, pl.ds(0, 16)] = jnp.where(lanes < TAIL_OFF, v, 0.0)
  for g in range(1, G // 16):
    ee2[TAIL_CH, pl.ds(g * 16, 16)] = zeros16


def _edge1_body(x_hbm, src3, dst3, als, ald, agg_out, den_out,
                als_v, ald_v, srcb, dst2, ee2, rows, zb, agg_s, den_s,
                sem):
  cid = lax.axis_index("c")
  sid = lax.axis_index("s")
  wid = cid * NS + sid
  zbase = sid * RPT
  zeros16 = jnp.zeros((16,), jnp.float32)

  _stage_common(als, ald, dst3, wid, als_v, ald_v, dst2)

  for g in range(G // 16):
    zb[pl.ds(g * 16, 16)] = zeros16

  def _zero_rows(e, _):
    for c in range(IN_CH // 16):
      rows[e, pl.ds(c * 16, 16)] = zeros16
    return 0
  lax.fori_loop(0, G, _zero_rows, 0)

  for k in range(RPT // G):
    pltpu.sync_copy(rows, agg_s.at[pl.ds(zbase + k * G, G)])
    pltpu.sync_copy(zb, den_s.at[pl.ds(zbase + k * G, G)])

  _compute_ee(src3, wid, als_v, ald_v, srcb, dst2, ee2)

  plsc.subcore_barrier()

  for j in range(NCH):
    pltpu.sync_copy(ee2.at[j], den_s.at[dst2.at[j]], add=True)

  def _scale(j):
    def _grp(m, _):
      coefs = ee2[j, pl.ds(m * 16, 16)]
      for l in range(16):
        cv = jnp.full((16,), coefs[l])
        e = m * 16 + l
        for c in range(IN_CH // 16):
          sl = pl.ds(c * 16, 16)
          rows[e, sl] = rows[e, sl] * cv
      return 0
    lax.fori_loop(0, G // 16, _grp, 0)

  pltpu.sync_copy(src3.at[wid, 0], srcb.at[0])
  pend = pltpu.async_copy(x_hbm.at[srcb.at[0]], rows, sem)
  for j in range(NCH):
    pend.wait()
    _scale(j)
    pltpu.sync_copy(rows, agg_s.at[dst2.at[j]], add=True)
    if j + 1 < NCH:
      pltpu.sync_copy(src3.at[wid, j + 1], srcb.at[0])
      pend = pltpu.async_copy(x_hbm.at[srcb.at[0]], rows, sem)
  plsc.subcore_barrier()

  for k in range(RPT // G):
    sl = pl.ds(zbase + k * G, G)
    pltpu.sync_copy(agg_s.at[sl], rows)
    pltpu.sync_copy(rows, agg_out.at[cid, sl])
    pltpu.sync_copy(den_s.at[sl], zb)
    pltpu.sync_copy(zb, den_out.at[cid, sl])


_edge_k1 = pl.kernel(
    _edge1_body,
    out_type=(
        jax.ShapeDtypeStruct((NC, NP, IN_CH), jnp.float32),
        jax.ShapeDtypeStruct((NC, NP), jnp.float32),
    ),
    mesh=_MESH,
    scratch_types=[
        pltpu.VMEM((NA,), jnp.float32),
        pltpu.VMEM((NA,), jnp.float32),
        pltpu.VMEM((1, G), jnp.int32),
        pltpu.VMEM((NCH, G), jnp.int32),
        pltpu.VMEM((NCH, G), jnp.float32),
        pltpu.VMEM((G, IN_CH), jnp.float32),
        pltpu.VMEM((G,), jnp.float32),
        pltpu.VMEM_SHARED((NP, IN_CH), jnp.float32),
        pltpu.VMEM_SHARED((NP,), jnp.float32),
        pltpu.SemaphoreType.DMA,
    ],
    compiler_params=_SC_PARAMS,
    name="gat_edge1",
)


def _edge2_body(h2f_hbm, src3, dst3, als, ald, agg_out, den_out,
                als_v, ald_v, src2, dst2, ee2, h2f_v, stage, zb,
                a0_s, a1_s, a2_s, a3_s, den_s):
  cid = lax.axis_index("c")
  sid = lax.axis_index("s")
  wid = cid * NS + sid
  zbase = sid * RPT
  zeros16 = jnp.zeros((16,), jnp.float32)
  accs = (a0_s, a1_s, a2_s, a3_s)

  _stage_common(als, ald, dst3, wid, als_v, ald_v, dst2)
  pltpu.sync_copy(src3.at[wid], src2)
  pltpu.sync_copy(h2f_hbm, h2f_v)

  for g in range(G // 16):
    zb[pl.ds(g * 16, 16)] = zeros16
  for k in range(RPT // G):
    sl = pl.ds(zbase + k * G, G)
    for acc in accs:
      pltpu.sync_copy(zb, acc.at[sl])
    pltpu.sync_copy(zb, den_s.at[sl])

  def _chunk(j, _):
    for m in range(G // 16):
      sl = pl.ds(m * 16, 16)
      s16 = src2[j, sl]
      d16 = dst2[j, sl]
      a = plsc.load_gather(als_v, [s16])
      b = plsc.load_gather(ald_v, [d16])
      e = a + b
      e = jnp.where(e > 0, e, 0.2 * e)
      ee2[j, sl] = jnp.exp(e)
    return 0
  lax.fori_loop(0, NCH, _chunk, 0)

  lanes = lax.iota(jnp.int32, 16)
  v = ee2[TAIL_CH, pl.ds(0, 16)]
  ee2[TAIL_CH, pl.ds(0, 16)] = jnp.where(lanes < TAIL_OFF, v, 0.0)
  for g in range(1, G // 16):
    ee2[TAIL_CH, pl.ds(g * 16, 16)] = zeros16

  plsc.subcore_barrier()

  for j in range(NCH):
    pltpu.sync_copy(ee2.at[j], den_s.at[dst2.at[j]], add=True)

  for j in range(NCH):
    def _grp_fixed(m, _):
      sl = pl.ds(m * 16, 16)
      s16 = src2[j, sl]
      ee16 = ee2[j, sl]
      for c in range(OUT_CH):
        vals = plsc.load_gather(h2f_v, [s16 + c * NP]) * ee16
        stage[pl.ds(c * G + m * 16, 16)] = vals
      return 0
    lax.fori_loop(0, G // 16, _grp_fixed, 0)
    for c in range(OUT_CH):
      pltpu.sync_copy(stage.at[pl.ds(c * G, G)],
                      accs[c].at[dst2.at[j]], add=True)
  plsc.subcore_barrier()

  for k in range(RPT // G):
    sl = pl.ds(zbase + k * G, G)
    for c in range(OUT_CH):
      pltpu.sync_copy(accs[c].at[sl], zb)
      pltpu.sync_copy(zb, agg_out.at[cid, c, sl])
    pltpu.sync_copy(den_s.at[sl], zb)
    pltpu.sync_copy(zb, den_out.at[cid, sl])


_edge_k2 = pl.kernel(
    _edge2_body,
    out_type=(
        jax.ShapeDtypeStruct((NC, OUT_CH, NP), jnp.float32),
        jax.ShapeDtypeStruct((NC, NP), jnp.float32),
    ),
    mesh=_MESH,
    scratch_types=[
        pltpu.VMEM((NA,), jnp.float32),
        pltpu.VMEM((NA,), jnp.float32),
        pltpu.VMEM((NCH, G), jnp.int32),
        pltpu.VMEM((NCH, G), jnp.int32),
        pltpu.VMEM((NCH, G), jnp.float32),
        pltpu.VMEM((OUT_CH * NP,), jnp.float32),
        pltpu.VMEM((OUT_CH * G,), jnp.float32),
        pltpu.VMEM((G,), jnp.float32),
        pltpu.VMEM_SHARED((NP,), jnp.float32),
        pltpu.VMEM_SHARED((NP,), jnp.float32),
        pltpu.VMEM_SHARED((NP,), jnp.float32),
        pltpu.VMEM_SHARED((NP,), jnp.float32),
        pltpu.VMEM_SHARED((NP,), jnp.float32),
    ],
    compiler_params=_SC_PARAMS,
    name="gat_edge2",
)


def _tc_pre1_body(x_ref, w_ref, as_ref, ad_ref, als_ref, ald_ref):
  wa = w_ref[...] @ jnp.concatenate([as_ref[...], ad_ref[...]], axis=1)
  al = x_ref[...] @ wa
  als_ref[...] = al[:, 0:1]
  ald_ref[...] = al[:, 1:2]


_tc_pre1 = pl.pallas_call(
    _tc_pre1_body,
    out_shape=(
        jax.ShapeDtypeStruct((N, 1), jnp.float32),
        jax.ShapeDtypeStruct((N, 1), jnp.float32),
    ),
)


def _tc_mid_body(agg_ref, den_ref, w1_ref, b1_ref, w2_ref, as_ref, ad_ref,
                 h2_ref, als_ref, ald_ref):
  agg = agg_ref[0] + agg_ref[1]
  den = jnp.maximum(den_ref[0] + den_ref[1], 1e-16)
  xs = agg[:N] * (1.0 / den[:N])[:, None]
  x2 = jax.nn.relu(xs @ w1_ref[...] + b1_ref[...])
  h2_ref[...] = x2 @ w2_ref[...]
  wa = w2_ref[...] @ jnp.concatenate([as_ref[...], ad_ref[...]], axis=1)
  al = x2 @ wa
  als_ref[...] = al[:, 0:1]
  ald_ref[...] = al[:, 1:2]


_tc_mid = pl.pallas_call(
    _tc_mid_body,
    out_shape=(
        jax.ShapeDtypeStruct((N, OUT_CH), jnp.float32),
        jax.ShapeDtypeStruct((N, 1), jnp.float32),
        jax.ShapeDtypeStruct((N, 1), jnp.float32),
    ),
)


def _tc_post_body(agg_ref, den_ref, b2_ref, out_ref):
  agg = agg_ref[0] + agg_ref[1]
  den = jnp.maximum(den_ref[0] + den_ref[1], 1e-16)
  out = agg[:, :N].T * (1.0 / den[:N])[:, None] + b2_ref[...]
  out_ref[...] = jax.nn.relu(out)


_tc_post = pl.pallas_call(
    _tc_post_body,
    out_shape=jax.ShapeDtypeStruct((N, OUT_CH), jnp.float32),
)


@jax.jit
def kernel(x, edge_index, W1, a_src1, a_dst1, b1, W2, a_src2, a_dst2, b2):
  src = edge_index[0].astype(jnp.int32)
  dst = edge_index[1].astype(jnp.int32)
  src3 = jnp.pad(src.reshape(NW, EPT), ((0, 0), (0, NCH * G - EPT))
                 ).reshape(NW, NCH, G)
  dst3 = jnp.pad(dst.reshape(NW, EPT), ((0, 0), (0, NCH * G - EPT))
                 ).reshape(NW, NCH, G)

  als1, ald1 = _tc_pre1(x, W1, a_src1.reshape(HID, 1),
                        a_dst1.reshape(HID, 1))
  agg1, den1 = _edge_k1(x, src3, dst3, als1.reshape(N), ald1.reshape(N))
  h2, als2, ald2 = _tc_mid(agg1, den1, W1, b1.reshape(1, HID), W2,
                           a_src2.reshape(OUT_CH, 1),
                           a_dst2.reshape(OUT_CH, 1))
  h2f = jnp.pad(h2.T, ((0, 0), (0, NP - N))).reshape(OUT_CH * NP)
  agg2, den2 = _edge_k2(h2f, src3, dst3, als2.reshape(N), ald2.reshape(N))
  return _tc_post(agg2, den2, b2.reshape(1, OUT_CH))


# speedup vs baseline: 23.0100x; 1.0881x over previous
import jax
import jax.numpy as jnp
from jax import lax
from jax.experimental import pallas as pl
from jax.experimental.pallas import tpu as pltpu
from jax.experimental.pallas import tpu_sc as plsc

N = 10000
NA = 10112
NP = 10240
E = 160000
IN_CH = 128
HID = 400
OUT_CH = 4
NC = 2
NS = 16
NW = NC * NS
EPT = E // NW
RPT = NP // NS
G = 128
NCH = -(-EPT // G)
TAIL_CH = EPT // G
TAIL_OFF = EPT - TAIL_CH * G

G1 = 64
NBUF = 2
NCB = 16
NBLK = 5
NCH1 = NCB * NBLK
NG1 = NCB // NBUF
TAIL_BCH1 = EPT // G1 - (NBLK - 1) * NCB
TAIL_OFF1 = EPT - (EPT // G1) * G1

_MESH = plsc.VectorSubcoreMesh(core_axis_name="c", subcore_axis_name="s")
_SC_PARAMS = pltpu.CompilerParams(needs_layout_passes=False)


def _stage_common(als, ald, dst3, wid, als_v, ald_v, dst2):
  pltpu.sync_copy(als, als_v)
  pltpu.sync_copy(ald, ald_v)
  pltpu.sync_copy(dst3.at[wid], dst2)


def _edge1_body(x_hbm, src3, dst3, als, ald, agg_out, den_out,
                src2, dst2, ee2, rows0, rows1, ab, zb,
                als_s, ald_s, agg_s, den_s, gs0, gs1, ss0, ss1):
  cid = lax.axis_index("c")
  sid = lax.axis_index("s")
  wid = cid * NS + sid
  zbase = sid * RPT
  zeros16 = jnp.zeros((16,), jnp.float32)
  rowsb = (rows0, rows1)
  gsems = (gs0, gs1)
  ssems = (ss0, ss1)

  @pl.when(sid == 0)
  def _():
    pltpu.sync_copy(als, als_s)
    pltpu.sync_copy(ald, ald_s)

  for g in range(G1 // 16):
    zb[pl.ds(g * 16, 16)] = zeros16

  def _zero_rows(e, _):
    for c in range(IN_CH // 16):
      rows0[e, pl.ds(c * 16, 16)] = zeros16
    return 0
  lax.fori_loop(0, G1, _zero_rows, 0)

  for k in range(RPT // G1):
    pltpu.sync_copy(rows0, agg_s.at[pl.ds(zbase + k * G1, G1)])
    pltpu.sync_copy(zb, den_s.at[pl.ds(zbase + k * G1, G1)])

  plsc.subcore_barrier()

  def _scale(jj, rows):
    def _grp(m, _):
      coefs = ee2[jj, pl.ds(m * 16, 16)]
      for l in range(16):
        cv = jnp.full((16,), coefs[l])
        for c in range(IN_CH // 16):
          sl = pl.ds(c * 16, 16)
          rows[l + m * 16, sl] = rows[l + m * 16, sl] * cv
      return 0
    lax.fori_loop(0, G1 // 16, _grp, 0)

  def _gcopy(jj, b):
    return pltpu.make_async_copy(x_hbm.at[src2.at[jj]], rowsb[b], gsems[b])

  def _scopy(jj, b):
    return pltpu.make_async_copy(rowsb[b], agg_s.at[dst2.at[jj]], ssems[b])

  lanes = lax.iota(jnp.int32, 16)

  for blk in range(NBLK):
    pltpu.sync_copy(src3.at[wid].at[pl.ds(blk * NCB, NCB)], src2)
    pltpu.sync_copy(dst3.at[wid].at[pl.ds(blk * NCB, NCB)], dst2)

    def _ee_chunk(j, _):
      pltpu.sync_copy(als_s.at[src2.at[j]], ab.at[0])
      pltpu.sync_copy(ald_s.at[dst2.at[j]], ab.at[1])
      for m in range(G1 // 16):
        sl = pl.ds(m * 16, 16)
        e = ab[0, sl] + ab[1, sl]
        e = jnp.where(e > 0, e, 0.2 * e)
        ee2[j, sl] = jnp.exp(e)
      return 0
    lax.fori_loop(0, NCB, _ee_chunk, 0)

    if blk == NBLK - 1:
      v = ee2[TAIL_BCH1, pl.ds(0, 16)]
      ee2[TAIL_BCH1, pl.ds(0, 16)] = jnp.where(lanes < TAIL_OFF1, v, 0.0)
      for g in range(1, G1 // 16):
        ee2[TAIL_BCH1, pl.ds(g * 16, 16)] = zeros16
      for j in range(TAIL_BCH1 + 1, NCB):
        for g in range(G1 // 16):
          ee2[j, pl.ds(g * 16, 16)] = zeros16

    def _den_chunk(j, _):
      pltpu.sync_copy(ee2.at[j], den_s.at[dst2.at[j]], add=True)
      return 0
    lax.fori_loop(0, NCB, _den_chunk, 0)

    for b in range(NBUF):
      _gcopy(b, b).start()

    def _super(g, _):
      base = g * NBUF
      for b in range(NBUF):
        jj = base + b
        _gcopy(jj, b).wait()
        _scale(jj, rowsb[b])
        _scopy(jj, b).start(add=True)
      nxt = jnp.minimum(base + NBUF, NCB - NBUF)
      for b in range(NBUF):
        jj = nxt + b
        _scopy(base + b, b).wait()
        _gcopy(jj, b).start()
      return 0
    lax.fori_loop(0, NG1 - 1, _super, 0)

    base = (NG1 - 1) * NBUF
    for b in range(NBUF):
      jj = base + b
      _gcopy(jj, b).wait()
      _scale(jj, rowsb[b])
      _scopy(jj, b).start(add=True)
    for b in range(NBUF):
      _scopy(base + b, b).wait()

  plsc.subcore_barrier()

  for k in range(RPT // G1):
    sl = pl.ds(zbase + k * G1, G1)
    pltpu.sync_copy(agg_s.at[sl], rows0)
    pltpu.sync_copy(rows0, agg_out.at[cid, sl])
    pltpu.sync_copy(den_s.at[sl], zb)
    pltpu.sync_copy(zb, den_out.at[cid, sl])


_edge_k1 = pl.kernel(
    _edge1_body,
    out_type=(
        jax.ShapeDtypeStruct((NC, NP, IN_CH), jnp.float32),
        jax.ShapeDtypeStruct((NC, NP), jnp.float32),
    ),
    mesh=_MESH,
    scratch_types=[
        pltpu.VMEM((NCB, G1), jnp.int32),
        pltpu.VMEM((NCB, G1), jnp.int32),
        pltpu.VMEM((NCB, G1), jnp.float32),
        pltpu.VMEM((G1, IN_CH), jnp.float32),
        pltpu.VMEM((G1, IN_CH), jnp.float32),
        pltpu.VMEM((2, G1), jnp.float32),
        pltpu.VMEM((G1,), jnp.float32),
        pltpu.VMEM_SHARED((NA,), jnp.float32),
        pltpu.VMEM_SHARED((NA,), jnp.float32),
        pltpu.VMEM_SHARED((NP, IN_CH), jnp.float32),
        pltpu.VMEM_SHARED((NP,), jnp.float32),
        pltpu.SemaphoreType.DMA,
        pltpu.SemaphoreType.DMA,
        pltpu.SemaphoreType.DMA,
        pltpu.SemaphoreType.DMA,
    ],
    compiler_params=_SC_PARAMS,
    name="gat_edge1",
)


def _edge2_body(h2f_hbm, src3, dst3, als, ald, agg_out, den_out,
                als_v, ald_v, src2, dst2, ee2, h2f_v, stage, zb,
                a0_s, a1_s, a2_s, a3_s, den_s):
  cid = lax.axis_index("c")
  sid = lax.axis_index("s")
  wid = cid * NS + sid
  zbase = sid * RPT
  zeros16 = jnp.zeros((16,), jnp.float32)
  accs = (a0_s, a1_s, a2_s, a3_s)

  _stage_common(als, ald, dst3, wid, als_v, ald_v, dst2)
  pltpu.sync_copy(src3.at[wid], src2)
  pltpu.sync_copy(h2f_hbm, h2f_v)

  for g in range(G // 16):
    zb[pl.ds(g * 16, 16)] = zeros16
  for k in range(RPT // G):
    sl = pl.ds(zbase + k * G, G)
    for acc in accs:
      pltpu.sync_copy(zb, acc.at[sl])
    pltpu.sync_copy(zb, den_s.at[sl])

  def _chunk(j, _):
    for m in range(G // 16):
      sl = pl.ds(m * 16, 16)
      s16 = src2[j, sl]
      d16 = dst2[j, sl]
      a = plsc.load_gather(als_v, [s16])
      b = plsc.load_gather(ald_v, [d16])
      e = a + b
      e = jnp.where(e > 0, e, 0.2 * e)
      ee2[j, sl] = jnp.exp(e)
    return 0
  lax.fori_loop(0, NCH, _chunk, 0)

  lanes = lax.iota(jnp.int32, 16)
  v = ee2[TAIL_CH, pl.ds(0, 16)]
  ee2[TAIL_CH, pl.ds(0, 16)] = jnp.where(lanes < TAIL_OFF, v, 0.0)
  for g in range(1, G // 16):
    ee2[TAIL_CH, pl.ds(g * 16, 16)] = zeros16

  plsc.subcore_barrier()

  for j in range(NCH):
    pltpu.sync_copy(ee2.at[j], den_s.at[dst2.at[j]], add=True)

  for j in range(NCH):
    def _grp_fixed(m, _):
      sl = pl.ds(m * 16, 16)
      s16 = src2[j, sl]
      ee16 = ee2[j, sl]
      for c in range(OUT_CH):
        vals = plsc.load_gather(h2f_v, [s16 + c * NP]) * ee16
        stage[pl.ds(c * G + m * 16, 16)] = vals
      return 0
    lax.fori_loop(0, G // 16, _grp_fixed, 0)
    for c in range(OUT_CH):
      pltpu.sync_copy(stage.at[pl.ds(c * G, G)],
                      accs[c].at[dst2.at[j]], add=True)
  plsc.subcore_barrier()

  for k in range(RPT // G):
    sl = pl.ds(zbase + k * G, G)
    for c in range(OUT_CH):
      pltpu.sync_copy(accs[c].at[sl], zb)
      pltpu.sync_copy(zb, agg_out.at[cid, c, sl])
    pltpu.sync_copy(den_s.at[sl], zb)
    pltpu.sync_copy(zb, den_out.at[cid, sl])


_edge_k2 = pl.kernel(
    _edge2_body,
    out_type=(
        jax.ShapeDtypeStruct((NC, OUT_CH, NP), jnp.float32),
        jax.ShapeDtypeStruct((NC, NP), jnp.float32),
    ),
    mesh=_MESH,
    scratch_types=[
        pltpu.VMEM((NA,), jnp.float32),
        pltpu.VMEM((NA,), jnp.float32),
        pltpu.VMEM((NCH, G), jnp.int32),
        pltpu.VMEM((NCH, G), jnp.int32),
        pltpu.VMEM((NCH, G), jnp.float32),
        pltpu.VMEM((OUT_CH * NP,), jnp.float32),
        pltpu.VMEM((OUT_CH * G,), jnp.float32),
        pltpu.VMEM((G,), jnp.float32),
        pltpu.VMEM_SHARED((NP,), jnp.float32),
        pltpu.VMEM_SHARED((NP,), jnp.float32),
        pltpu.VMEM_SHARED((NP,), jnp.float32),
        pltpu.VMEM_SHARED((NP,), jnp.float32),
        pltpu.VMEM_SHARED((NP,), jnp.float32),
    ],
    compiler_params=_SC_PARAMS,
    name="gat_edge2",
)


def _tc_pre1_body(x_ref, w_ref, as_ref, ad_ref, als_ref, ald_ref):
  wa = w_ref[...] @ jnp.concatenate([as_ref[...], ad_ref[...]], axis=1)
  al = x_ref[...] @ wa
  als_ref[...] = al[:, 0:1]
  ald_ref[...] = al[:, 1:2]


_tc_pre1 = pl.pallas_call(
    _tc_pre1_body,
    out_shape=(
        jax.ShapeDtypeStruct((N, 1), jnp.float32),
        jax.ShapeDtypeStruct((N, 1), jnp.float32),
    ),
)


def _tc_mid_body(agg_ref, den_ref, w1_ref, b1_ref, w2_ref, as_ref, ad_ref,
                 h2_ref, als_ref, ald_ref):
  agg = agg_ref[0] + agg_ref[1]
  den = jnp.maximum(den_ref[0] + den_ref[1], 1e-16)
  xs = agg[:N] * (1.0 / den[:N])[:, None]
  x2 = jax.nn.relu(xs @ w1_ref[...] + b1_ref[...])
  h2_ref[...] = x2 @ w2_ref[...]
  wa = w2_ref[...] @ jnp.concatenate([as_ref[...], ad_ref[...]], axis=1)
  al = x2 @ wa
  als_ref[...] = al[:, 0:1]
  ald_ref[...] = al[:, 1:2]


_tc_mid = pl.pallas_call(
    _tc_mid_body,
    out_shape=(
        jax.ShapeDtypeStruct((N, OUT_CH), jnp.float32),
        jax.ShapeDtypeStruct((N, 1), jnp.float32),
        jax.ShapeDtypeStruct((N, 1), jnp.float32),
    ),
)


def _tc_post_body(agg_ref, den_ref, b2_ref, out_ref):
  agg = agg_ref[0] + agg_ref[1]
  den = jnp.maximum(den_ref[0] + den_ref[1], 1e-16)
  out = agg[:, :N].T * (1.0 / den[:N])[:, None] + b2_ref[...]
  out_ref[...] = jax.nn.relu(out)


_tc_post = pl.pallas_call(
    _tc_post_body,
    out_shape=jax.ShapeDtypeStruct((N, OUT_CH), jnp.float32),
)


@jax.jit
def kernel(x, edge_index, W1, a_src1, a_dst1, b1, W2, a_src2, a_dst2, b2):
  src = edge_index[0].astype(jnp.int32)
  dst = edge_index[1].astype(jnp.int32)
  src3a = jnp.pad(src.reshape(NW, EPT), ((0, 0), (0, NCH1 * G1 - EPT))
                  ).reshape(NW, NCH1, G1)
  dst3a = jnp.pad(dst.reshape(NW, EPT), ((0, 0), (0, NCH1 * G1 - EPT))
                  ).reshape(NW, NCH1, G1)
  src3 = jnp.pad(src.reshape(NW, EPT), ((0, 0), (0, NCH * G - EPT))
                 ).reshape(NW, NCH, G)
  dst3 = jnp.pad(dst.reshape(NW, EPT), ((0, 0), (0, NCH * G - EPT))
                 ).reshape(NW, NCH, G)

  als1, ald1 = _tc_pre1(x, W1, a_src1.reshape(HID, 1),
                        a_dst1.reshape(HID, 1))
  als1p = jnp.pad(als1.reshape(N), (0, NA - N))
  ald1p = jnp.pad(ald1.reshape(N), (0, NA - N))
  agg1, den1 = _edge_k1(x, src3a, dst3a, als1p, ald1p)
  h2, als2, ald2 = _tc_mid(agg1, den1, W1, b1.reshape(1, HID), W2,
                           a_src2.reshape(OUT_CH, 1),
                           a_dst2.reshape(OUT_CH, 1))
  h2f = jnp.pad(h2.T, ((0, 0), (0, NP - N))).reshape(OUT_CH * NP)
  als2p = jnp.pad(als2.reshape(N), (0, NA - N))
  ald2p = jnp.pad(ald2.reshape(N), (0, NA - N))
  agg2, den2 = _edge_k2(h2f, src3, dst3, als2p, ald2p)
  return _tc_post(agg2, den2, b2.reshape(1, OUT_CH))


# speedup vs baseline: 23.7998x; 1.0343x over previous
import jax
import jax.numpy as jnp
from jax import lax
from jax.experimental import pallas as pl
from jax.experimental.pallas import tpu as pltpu
from jax.experimental.pallas import tpu_sc as plsc

N = 10000
NA = 10112
NP = 10240
E = 160000
IN_CH = 128
HID = 400
OUT_CH = 4
NC = 2
NS = 16
NW = NC * NS
EPT = E // NW
RPT = NP // NS
G = 128
NCH = -(-EPT // G)
TAIL_CH = EPT // G
TAIL_OFF = EPT - TAIL_CH * G

G1 = 64
NBUF = 2
NCB = 16
NBLK = 5
NCH1 = NCB * NBLK
NG1 = NCB // NBUF
TAIL_BCH1 = EPT // G1 - (NBLK - 1) * NCB
TAIL_OFF1 = EPT - (EPT // G1) * G1

_MESH = plsc.VectorSubcoreMesh(core_axis_name="c", subcore_axis_name="s")
_SC_PARAMS = pltpu.CompilerParams(needs_layout_passes=False)


def _stage_common(als, ald, dst3, wid, als_v, ald_v, dst2):
  pltpu.sync_copy(als, als_v)
  pltpu.sync_copy(ald, ald_v)
  pltpu.sync_copy(dst3.at[wid], dst2)


def _edge1_body(x_hbm, src3, dst3, als, ald, agg_out, den_out,
                src2, dst2, ee2, rows0, rows1, ab, zb,
                als_s, ald_s, agg_s, den_s, gs0, gs1, ss0, ss1,
                as0, as1, dsem):
  cid = lax.axis_index("c")
  sid = lax.axis_index("s")
  wid = cid * NS + sid
  zbase = sid * RPT
  zeros16 = jnp.zeros((16,), jnp.float32)
  rowsb = (rows0, rows1)
  gsems = (gs0, gs1)
  ssems = (ss0, ss1)

  @pl.when(sid == 0)
  def _():
    pltpu.sync_copy(als, als_s)
    pltpu.sync_copy(ald, ald_s)

  for g in range(G1 // 16):
    zb[pl.ds(g * 16, 16)] = zeros16

  def _zero_rows(e, _):
    for c in range(IN_CH // 16):
      rows0[e, pl.ds(c * 16, 16)] = zeros16
    return 0
  lax.fori_loop(0, G1, _zero_rows, 0)

  for k in range(RPT // G1):
    pltpu.sync_copy(rows0, agg_s.at[pl.ds(zbase + k * G1, G1)])
    pltpu.sync_copy(zb, den_s.at[pl.ds(zbase + k * G1, G1)])

  plsc.subcore_barrier()

  def _scale(jj, rows):
    def _grp(m, _):
      coefs = ee2[jj, pl.ds(m * 16, 16)]
      for l in range(16):
        cv = jnp.full((16,), coefs[l])
        for c in range(IN_CH // 16):
          sl = pl.ds(c * 16, 16)
          rows[l + m * 16, sl] = rows[l + m * 16, sl] * cv
      return 0
    lax.fori_loop(0, G1 // 16, _grp, 0)

  def _gcopy(jj, b):
    return pltpu.make_async_copy(x_hbm.at[src2.at[jj]], rowsb[b], gsems[b])

  def _scopy(jj, b):
    return pltpu.make_async_copy(rowsb[b], agg_s.at[dst2.at[jj]], ssems[b])

  lanes = lax.iota(jnp.int32, 16)

  for blk in range(NBLK):
    pltpu.sync_copy(src3.at[wid].at[pl.ds(blk * NCB, NCB)], src2)
    pltpu.sync_copy(dst3.at[wid].at[pl.ds(blk * NCB, NCB)], dst2)

    def _ee_chunk(j, _):
      ca = pltpu.make_async_copy(als_s.at[src2.at[j]], ab.at[0], as0)
      cb = pltpu.make_async_copy(ald_s.at[dst2.at[j]], ab.at[1], as1)
      ca.start()
      cb.start()
      ca.wait()
      cb.wait()
      for m in range(G1 // 16):
        sl = pl.ds(m * 16, 16)
        e = ab[0, sl] + ab[1, sl]
        e = jnp.where(e > 0, e, 0.2 * e)
        ee2[j, sl] = jnp.exp(e)
      return 0
    lax.fori_loop(0, NCB, _ee_chunk, 0)

    if blk == NBLK - 1:
      v = ee2[TAIL_BCH1, pl.ds(0, 16)]
      ee2[TAIL_BCH1, pl.ds(0, 16)] = jnp.where(lanes < TAIL_OFF1, v, 0.0)
      for g in range(1, G1 // 16):
        ee2[TAIL_BCH1, pl.ds(g * 16, 16)] = zeros16
      for j in range(TAIL_BCH1 + 1, NCB):
        for g in range(G1 // 16):
          ee2[j, pl.ds(g * 16, 16)] = zeros16

    def _dcopy(j):
      return pltpu.make_async_copy(ee2.at[j], den_s.at[dst2.at[j]], dsem)

    def _den_start(j, _):
      _dcopy(j).start(add=True)
      return 0
    lax.fori_loop(0, NCB, _den_start, 0)

    for b in range(NBUF):
      _gcopy(b, b).start()

    def _super(g, _):
      base = g * NBUF
      for b in range(NBUF):
        jj = base + b
        _gcopy(jj, b).wait()
        _scale(jj, rowsb[b])
        _scopy(jj, b).start(add=True)
      nxt = jnp.minimum(base + NBUF, NCB - NBUF)
      for b in range(NBUF):
        jj = nxt + b
        _scopy(base + b, b).wait()
        _gcopy(jj, b).start()
      return 0
    lax.fori_loop(0, NG1 - 1, _super, 0)

    base = (NG1 - 1) * NBUF
    for b in range(NBUF):
      jj = base + b
      _gcopy(jj, b).wait()
      _scale(jj, rowsb[b])
      _scopy(jj, b).start(add=True)
    for b in range(NBUF):
      _scopy(base + b, b).wait()

    def _den_wait(j, _):
      _dcopy(j).wait()
      return 0
    lax.fori_loop(0, NCB, _den_wait, 0)

  plsc.subcore_barrier()

  for k in range(RPT // G1):
    sl = pl.ds(zbase + k * G1, G1)
    pltpu.sync_copy(agg_s.at[sl], rows0)
    pltpu.sync_copy(rows0, agg_out.at[cid, sl])
    pltpu.sync_copy(den_s.at[sl], zb)
    pltpu.sync_copy(zb, den_out.at[cid, sl])


_edge_k1 = pl.kernel(
    _edge1_body,
    out_type=(
        jax.ShapeDtypeStruct((NC, NP, IN_CH), jnp.float32),
        jax.ShapeDtypeStruct((NC, NP), jnp.float32),
    ),
    mesh=_MESH,
    scratch_types=[
        pltpu.VMEM((NCB, G1), jnp.int32),
        pltpu.VMEM((NCB, G1), jnp.int32),
        pltpu.VMEM((NCB, G1), jnp.float32),
        pltpu.VMEM((G1, IN_CH), jnp.float32),
        pltpu.VMEM((G1, IN_CH), jnp.float32),
        pltpu.VMEM((2, G1), jnp.float32),
        pltpu.VMEM((G1,), jnp.float32),
        pltpu.VMEM_SHARED((NA,), jnp.float32),
        pltpu.VMEM_SHARED((NA,), jnp.float32),
        pltpu.VMEM_SHARED((NP, IN_CH), jnp.float32),
        pltpu.VMEM_SHARED((NP,), jnp.float32),
        pltpu.SemaphoreType.DMA,
        pltpu.SemaphoreType.DMA,
        pltpu.SemaphoreType.DMA,
        pltpu.SemaphoreType.DMA,
        pltpu.SemaphoreType.DMA,
        pltpu.SemaphoreType.DMA,
        pltpu.SemaphoreType.DMA,
    ],
    compiler_params=_SC_PARAMS,
    name="gat_edge1",
)


def _edge2_body(h2f_hbm, src3, dst3, als, ald, agg_out, den_out,
                als_v, ald_v, src2, dst2, ee2, h2f_v, stage, zb,
                a0_s, a1_s, a2_s, a3_s, den_s):
  cid = lax.axis_index("c")
  sid = lax.axis_index("s")
  wid = cid * NS + sid
  zbase = sid * RPT
  zeros16 = jnp.zeros((16,), jnp.float32)
  accs = (a0_s, a1_s, a2_s, a3_s)

  _stage_common(als, ald, dst3, wid, als_v, ald_v, dst2)
  pltpu.sync_copy(src3.at[wid], src2)
  pltpu.sync_copy(h2f_hbm, h2f_v)

  for g in range(G // 16):
    zb[pl.ds(g * 16, 16)] = zeros16
  for k in range(RPT // G):
    sl = pl.ds(zbase + k * G, G)
    for acc in accs:
      pltpu.sync_copy(zb, acc.at[sl])
    pltpu.sync_copy(zb, den_s.at[sl])

  def _chunk(j, _):
    for m in range(G // 16):
      sl = pl.ds(m * 16, 16)
      s16 = src2[j, sl]
      d16 = dst2[j, sl]
      a = plsc.load_gather(als_v, [s16])
      b = plsc.load_gather(ald_v, [d16])
      e = a + b
      e = jnp.where(e > 0, e, 0.2 * e)
      ee2[j, sl] = jnp.exp(e)
    return 0
  lax.fori_loop(0, NCH, _chunk, 0)

  lanes = lax.iota(jnp.int32, 16)
  v = ee2[TAIL_CH, pl.ds(0, 16)]
  ee2[TAIL_CH, pl.ds(0, 16)] = jnp.where(lanes < TAIL_OFF, v, 0.0)
  for g in range(1, G // 16):
    ee2[TAIL_CH, pl.ds(g * 16, 16)] = zeros16

  plsc.subcore_barrier()

  for j in range(NCH):
    pltpu.sync_copy(ee2.at[j], den_s.at[dst2.at[j]], add=True)

  for j in range(NCH):
    def _grp_fixed(m, _):
      sl = pl.ds(m * 16, 16)
      s16 = src2[j, sl]
      ee16 = ee2[j, sl]
      for c in range(OUT_CH):
        vals = plsc.load_gather(h2f_v, [s16 + c * NP]) * ee16
        stage[pl.ds(c * G + m * 16, 16)] = vals
      return 0
    lax.fori_loop(0, G // 16, _grp_fixed, 0)
    for c in range(OUT_CH):
      pltpu.sync_copy(stage.at[pl.ds(c * G, G)],
                      accs[c].at[dst2.at[j]], add=True)
  plsc.subcore_barrier()

  for k in range(RPT // G):
    sl = pl.ds(zbase + k * G, G)
    for c in range(OUT_CH):
      pltpu.sync_copy(accs[c].at[sl], zb)
      pltpu.sync_copy(zb, agg_out.at[cid, c, sl])
    pltpu.sync_copy(den_s.at[sl], zb)
    pltpu.sync_copy(zb, den_out.at[cid, sl])


_edge_k2 = pl.kernel(
    _edge2_body,
    out_type=(
        jax.ShapeDtypeStruct((NC, OUT_CH, NP), jnp.float32),
        jax.ShapeDtypeStruct((NC, NP), jnp.float32),
    ),
    mesh=_MESH,
    scratch_types=[
        pltpu.VMEM((NA,), jnp.float32),
        pltpu.VMEM((NA,), jnp.float32),
        pltpu.VMEM((NCH, G), jnp.int32),
        pltpu.VMEM((NCH, G), jnp.int32),
        pltpu.VMEM((NCH, G), jnp.float32),
        pltpu.VMEM((OUT_CH * NP,), jnp.float32),
        pltpu.VMEM((OUT_CH * G,), jnp.float32),
        pltpu.VMEM((G,), jnp.float32),
        pltpu.VMEM_SHARED((NP,), jnp.float32),
        pltpu.VMEM_SHARED((NP,), jnp.float32),
        pltpu.VMEM_SHARED((NP,), jnp.float32),
        pltpu.VMEM_SHARED((NP,), jnp.float32),
        pltpu.VMEM_SHARED((NP,), jnp.float32),
    ],
    compiler_params=_SC_PARAMS,
    name="gat_edge2",
)


def _tc_pre1_body(x_ref, w_ref, as_ref, ad_ref, als_ref, ald_ref):
  wa = w_ref[...] @ jnp.concatenate([as_ref[...], ad_ref[...]], axis=1)
  al = x_ref[...] @ wa
  als_ref[...] = al[:, 0:1]
  ald_ref[...] = al[:, 1:2]


_tc_pre1 = pl.pallas_call(
    _tc_pre1_body,
    out_shape=(
        jax.ShapeDtypeStruct((N, 1), jnp.float32),
        jax.ShapeDtypeStruct((N, 1), jnp.float32),
    ),
)


def _tc_mid_body(agg_ref, den_ref, w1_ref, b1_ref, w2_ref, as_ref, ad_ref,
                 h2_ref, als_ref, ald_ref):
  agg = agg_ref[0] + agg_ref[1]
  den = jnp.maximum(den_ref[0] + den_ref[1], 1e-16)
  xs = agg[:N] * (1.0 / den[:N])[:, None]
  x2 = jax.nn.relu(xs @ w1_ref[...] + b1_ref[...])
  h2_ref[...] = x2 @ w2_ref[...]
  wa = w2_ref[...] @ jnp.concatenate([as_ref[...], ad_ref[...]], axis=1)
  al = x2 @ wa
  als_ref[...] = al[:, 0:1]
  ald_ref[...] = al[:, 1:2]


_tc_mid = pl.pallas_call(
    _tc_mid_body,
    out_shape=(
        jax.ShapeDtypeStruct((N, OUT_CH), jnp.float32),
        jax.ShapeDtypeStruct((N, 1), jnp.float32),
        jax.ShapeDtypeStruct((N, 1), jnp.float32),
    ),
)


def _tc_post_body(agg_ref, den_ref, b2_ref, out_ref):
  agg = agg_ref[0] + agg_ref[1]
  den = jnp.maximum(den_ref[0] + den_ref[1], 1e-16)
  out = agg[:, :N].T * (1.0 / den[:N])[:, None] + b2_ref[...]
  out_ref[...] = jax.nn.relu(out)


_tc_post = pl.pallas_call(
    _tc_post_body,
    out_shape=jax.ShapeDtypeStruct((N, OUT_CH), jnp.float32),
)


@jax.jit
def kernel(x, edge_index, W1, a_src1, a_dst1, b1, W2, a_src2, a_dst2, b2):
  src = edge_index[0].astype(jnp.int32)
  dst = edge_index[1].astype(jnp.int32)
  src3a = jnp.pad(src.reshape(NW, EPT), ((0, 0), (0, NCH1 * G1 - EPT))
                  ).reshape(NW, NCH1, G1)
  dst3a = jnp.pad(dst.reshape(NW, EPT), ((0, 0), (0, NCH1 * G1 - EPT))
                  ).reshape(NW, NCH1, G1)
  src3 = jnp.pad(src.reshape(NW, EPT), ((0, 0), (0, NCH * G - EPT))
                 ).reshape(NW, NCH, G)
  dst3 = jnp.pad(dst.reshape(NW, EPT), ((0, 0), (0, NCH * G - EPT))
                 ).reshape(NW, NCH, G)

  als1, ald1 = _tc_pre1(x, W1, a_src1.reshape(HID, 1),
                        a_dst1.reshape(HID, 1))
  als1p = jnp.pad(als1.reshape(N), (0, NA - N))
  ald1p = jnp.pad(ald1.reshape(N), (0, NA - N))
  agg1, den1 = _edge_k1(x, src3a, dst3a, als1p, ald1p)
  h2, als2, ald2 = _tc_mid(agg1, den1, W1, b1.reshape(1, HID), W2,
                           a_src2.reshape(OUT_CH, 1),
                           a_dst2.reshape(OUT_CH, 1))
  h2f = jnp.pad(h2.T, ((0, 0), (0, NP - N))).reshape(OUT_CH * NP)
  als2p = jnp.pad(als2.reshape(N), (0, NA - N))
  ald2p = jnp.pad(ald2.reshape(N), (0, NA - N))
  agg2, den2 = _edge_k2(h2f, src3, dst3, als2p, ald2p)
  return _tc_post(agg2, den2, b2.reshape(1, OUT_CH))


# speedup vs baseline: 23.9372x; 1.0058x over previous
import jax
import jax.numpy as jnp
from jax import lax
from jax.experimental import pallas as pl
from jax.experimental.pallas import tpu as pltpu
from jax.experimental.pallas import tpu_sc as plsc

N = 10000
NA = 10112
NP = 10240
E = 160000
IN_CH = 128
HID = 400
OUT_CH = 4
NC = 2
NS = 16
NW = NC * NS
EPT = E // NW
RPT = NP // NS
G = 128
NCH = -(-EPT // G)
TAIL_CH = EPT // G
TAIL_OFF = EPT - TAIL_CH * G

G1 = 64
NBUF = 2
NCB = 16
NBLK = 5
NCH1 = NCB * NBLK
NG1 = NCB // NBUF
TAIL_BCH1 = EPT // G1 - (NBLK - 1) * NCB
TAIL_OFF1 = EPT - (EPT // G1) * G1

_MESH = plsc.VectorSubcoreMesh(core_axis_name="c", subcore_axis_name="s")
_SC_PARAMS = pltpu.CompilerParams(needs_layout_passes=False)


def _stage_common(als, ald, dst3, wid, als_v, ald_v, dst2):
  pltpu.sync_copy(als, als_v)
  pltpu.sync_copy(ald, ald_v)
  pltpu.sync_copy(dst3.at[wid], dst2)


def _edge1_body(x_hbm, src3, dst3, als, ald, agg_out, den_out,
                src2, dst2, ee2, rows0, rows1, ab, zb,
                als_s, ald_s, agg_s, den_s, gs0, gs1, ss0, ss1,
                as0, as1, dsem):
  cid = lax.axis_index("c")
  sid = lax.axis_index("s")
  wid = cid * NS + sid
  zbase = sid * RPT
  zeros16 = jnp.zeros((16,), jnp.float32)
  rowsb = (rows0, rows1)
  gsems = (gs0, gs1)
  ssems = (ss0, ss1)

  @pl.when(sid == 0)
  def _():
    pltpu.sync_copy(als, als_s)
    pltpu.sync_copy(ald, ald_s)

  for g in range(G1 // 16):
    zb[pl.ds(g * 16, 16)] = zeros16

  def _zero_rows(e, _):
    for c in range(IN_CH // 16):
      rows0[e, pl.ds(c * 16, 16)] = zeros16
    return 0
  lax.fori_loop(0, G1, _zero_rows, 0)

  for k in range(RPT // G1):
    pltpu.sync_copy(rows0, agg_s.at[pl.ds(zbase + k * G1, G1)])
    pltpu.sync_copy(zb, den_s.at[pl.ds(zbase + k * G1, G1)])

  plsc.subcore_barrier()

  def _scale(jj, rows):
    def _grp(m, _):
      coefs = ee2[jj, pl.ds(m * 16, 16)]
      for l in range(16):
        cv = jnp.full((16,), coefs[l])
        for c in range(IN_CH // 16):
          sl = pl.ds(c * 16, 16)
          rows[l + m * 16, sl] = rows[l + m * 16, sl] * cv
      return 0
    lax.fori_loop(0, G1 // 16, _grp, 0)

  def _gcopy(jj, b):
    return pltpu.make_async_copy(x_hbm.at[src2.at[jj]], rowsb[b], gsems[b])

  def _scopy(jj, b):
    return pltpu.make_async_copy(rowsb[b], agg_s.at[dst2.at[jj]], ssems[b])

  lanes = lax.iota(jnp.int32, 16)

  for blk in range(NBLK):
    pltpu.sync_copy(src3.at[wid].at[pl.ds(blk * NCB, NCB)], src2)
    pltpu.sync_copy(dst3.at[wid].at[pl.ds(blk * NCB, NCB)], dst2)

    def _ee_chunk(j, _):
      ca = pltpu.make_async_copy(als_s.at[src2.at[j]], ab.at[0], as0)
      cb = pltpu.make_async_copy(ald_s.at[dst2.at[j]], ab.at[1], as1)
      ca.start()
      cb.start()
      ca.wait()
      cb.wait()
      for m in range(G1 // 16):
        sl = pl.ds(m * 16, 16)
        e = ab[0, sl] + ab[1, sl]
        e = jnp.where(e > 0, e, 0.2 * e)
        ee2[j, sl] = jnp.exp(e)
      return 0
    lax.fori_loop(0, NCB, _ee_chunk, 0)

    if blk == NBLK - 1:
      v = ee2[TAIL_BCH1, pl.ds(0, 16)]
      ee2[TAIL_BCH1, pl.ds(0, 16)] = jnp.where(lanes < TAIL_OFF1, v, 0.0)
      for g in range(1, G1 // 16):
        ee2[TAIL_BCH1, pl.ds(g * 16, 16)] = zeros16
      for j in range(TAIL_BCH1 + 1, NCB):
        for g in range(G1 // 16):
          ee2[j, pl.ds(g * 16, 16)] = zeros16

    def _dcopy(j):
      return pltpu.make_async_copy(ee2.at[j], den_s.at[dst2.at[j]], dsem)

    def _den_start(j, _):
      _dcopy(j).start(add=True)
      return 0
    lax.fori_loop(0, NCB, _den_start, 0)

    for b in range(NBUF):
      _gcopy(b, b).start()

    def _super(g, _):
      base = g * NBUF
      for b in range(NBUF):
        jj = base + b
        _gcopy(jj, b).wait()
        _scale(jj, rowsb[b])
        _scopy(jj, b).start(add=True)
      nxt = jnp.minimum(base + NBUF, NCB - NBUF)
      for b in range(NBUF):
        jj = nxt + b
        _scopy(base + b, b).wait()
        _gcopy(jj, b).start()
      return 0
    lax.fori_loop(0, NG1 - 1, _super, 0)

    base = (NG1 - 1) * NBUF
    for b in range(NBUF):
      jj = base + b
      _gcopy(jj, b).wait()
      _scale(jj, rowsb[b])
      _scopy(jj, b).start(add=True)
    for b in range(NBUF):
      _scopy(base + b, b).wait()

    def _den_wait(j, _):
      _dcopy(j).wait()
      return 0
    lax.fori_loop(0, NCB, _den_wait, 0)

  plsc.subcore_barrier()

  for k in range(RPT // G1):
    sl = pl.ds(zbase + k * G1, G1)
    pltpu.sync_copy(agg_s.at[sl], rows0)
    pltpu.sync_copy(rows0, agg_out.at[cid, sl])
    pltpu.sync_copy(den_s.at[sl], zb)
    pltpu.sync_copy(zb, den_out.at[cid, sl])


_edge_k1 = pl.kernel(
    _edge1_body,
    out_type=(
        jax.ShapeDtypeStruct((NC, NP, IN_CH), jnp.float32),
        jax.ShapeDtypeStruct((NC, NP), jnp.float32),
    ),
    mesh=_MESH,
    scratch_types=[
        pltpu.VMEM((NCB, G1), jnp.int32),
        pltpu.VMEM((NCB, G1), jnp.int32),
        pltpu.VMEM((NCB, G1), jnp.float32),
        pltpu.VMEM((G1, IN_CH), jnp.float32),
        pltpu.VMEM((G1, IN_CH), jnp.float32),
        pltpu.VMEM((2, G1), jnp.float32),
        pltpu.VMEM((G1,), jnp.float32),
        pltpu.VMEM_SHARED((NA,), jnp.float32),
        pltpu.VMEM_SHARED((NA,), jnp.float32),
        pltpu.VMEM_SHARED((NP, IN_CH), jnp.float32),
        pltpu.VMEM_SHARED((NP,), jnp.float32),
        pltpu.SemaphoreType.DMA,
        pltpu.SemaphoreType.DMA,
        pltpu.SemaphoreType.DMA,
        pltpu.SemaphoreType.DMA,
        pltpu.SemaphoreType.DMA,
        pltpu.SemaphoreType.DMA,
        pltpu.SemaphoreType.DMA,
    ],
    compiler_params=_SC_PARAMS,
    name="gat_edge1",
)


def _edge2_body(h2f_hbm, src3, dst3, als, ald, agg_out, den_out,
                als_v, ald_v, src2, dst2, ee2, h2f_v, stage, zb,
                a0_s, a1_s, a2_s, a3_s, den_s, dsem):
  cid = lax.axis_index("c")
  sid = lax.axis_index("s")
  wid = cid * NS + sid
  zbase = sid * RPT
  zeros16 = jnp.zeros((16,), jnp.float32)
  accs = (a0_s, a1_s, a2_s, a3_s)

  _stage_common(als, ald, dst3, wid, als_v, ald_v, dst2)
  pltpu.sync_copy(src3.at[wid], src2)
  pltpu.sync_copy(h2f_hbm, h2f_v)

  for g in range(G // 16):
    zb[pl.ds(g * 16, 16)] = zeros16
  for k in range(RPT // G):
    sl = pl.ds(zbase + k * G, G)
    for acc in accs:
      pltpu.sync_copy(zb, acc.at[sl])
    pltpu.sync_copy(zb, den_s.at[sl])

  def _chunk(j, _):
    for m in range(G // 16):
      sl = pl.ds(m * 16, 16)
      s16 = src2[j, sl]
      d16 = dst2[j, sl]
      a = plsc.load_gather(als_v, [s16])
      b = plsc.load_gather(ald_v, [d16])
      e = a + b
      e = jnp.where(e > 0, e, 0.2 * e)
      ee2[j, sl] = jnp.exp(e)
    return 0
  lax.fori_loop(0, NCH, _chunk, 0)

  lanes = lax.iota(jnp.int32, 16)
  v = ee2[TAIL_CH, pl.ds(0, 16)]
  ee2[TAIL_CH, pl.ds(0, 16)] = jnp.where(lanes < TAIL_OFF, v, 0.0)
  for g in range(1, G // 16):
    ee2[TAIL_CH, pl.ds(g * 16, 16)] = zeros16

  plsc.subcore_barrier()

  def _dcopy(j):
    return pltpu.make_async_copy(ee2.at[j], den_s.at[dst2.at[j]], dsem)

  for j in range(NCH):
    _dcopy(j).start(add=True)

  for j in range(NCH):
    def _grp_fixed(m, _):
      sl = pl.ds(m * 16, 16)
      s16 = src2[j, sl]
      ee16 = ee2[j, sl]
      for c in range(OUT_CH):
        vals = plsc.load_gather(h2f_v, [s16 + c * NP]) * ee16
        stage[pl.ds(c * G + m * 16, 16)] = vals
      return 0
    lax.fori_loop(0, G // 16, _grp_fixed, 0)
    for c in range(OUT_CH):
      pltpu.sync_copy(stage.at[pl.ds(c * G, G)],
                      accs[c].at[dst2.at[j]], add=True)
  for j in range(NCH):
    _dcopy(j).wait()
  plsc.subcore_barrier()

  for k in range(RPT // G):
    sl = pl.ds(zbase + k * G, G)
    for c in range(OUT_CH):
      pltpu.sync_copy(accs[c].at[sl], zb)
      pltpu.sync_copy(zb, agg_out.at[cid, c, sl])
    pltpu.sync_copy(den_s.at[sl], zb)
    pltpu.sync_copy(zb, den_out.at[cid, sl])


_edge_k2 = pl.kernel(
    _edge2_body,
    out_type=(
        jax.ShapeDtypeStruct((NC, OUT_CH, NP), jnp.float32),
        jax.ShapeDtypeStruct((NC, NP), jnp.float32),
    ),
    mesh=_MESH,
    scratch_types=[
        pltpu.VMEM((NA,), jnp.float32),
        pltpu.VMEM((NA,), jnp.float32),
        pltpu.VMEM((NCH, G), jnp.int32),
        pltpu.VMEM((NCH, G), jnp.int32),
        pltpu.VMEM((NCH, G), jnp.float32),
        pltpu.VMEM((OUT_CH * NP,), jnp.float32),
        pltpu.VMEM((OUT_CH * G,), jnp.float32),
        pltpu.VMEM((G,), jnp.float32),
        pltpu.VMEM_SHARED((NP,), jnp.float32),
        pltpu.VMEM_SHARED((NP,), jnp.float32),
        pltpu.VMEM_SHARED((NP,), jnp.float32),
        pltpu.VMEM_SHARED((NP,), jnp.float32),
        pltpu.VMEM_SHARED((NP,), jnp.float32),
        pltpu.SemaphoreType.DMA,
    ],
    compiler_params=_SC_PARAMS,
    name="gat_edge2",
)


def _tc_pre1_body(x_ref, w_ref, as_ref, ad_ref, als_ref, ald_ref):
  wa = w_ref[...] @ jnp.concatenate([as_ref[...], ad_ref[...]], axis=1)
  al = x_ref[...] @ wa
  als_ref[...] = al[:, 0:1]
  ald_ref[...] = al[:, 1:2]


_tc_pre1 = pl.pallas_call(
    _tc_pre1_body,
    out_shape=(
        jax.ShapeDtypeStruct((N, 1), jnp.float32),
        jax.ShapeDtypeStruct((N, 1), jnp.float32),
    ),
)


def _tc_mid_body(agg_ref, den_ref, w1_ref, b1_ref, w2_ref, as_ref, ad_ref,
                 h2_ref, als_ref, ald_ref):
  agg = agg_ref[0] + agg_ref[1]
  den = jnp.maximum(den_ref[0] + den_ref[1], 1e-16)
  xs = agg[:N] * (1.0 / den[:N])[:, None]
  x2 = jax.nn.relu(xs @ w1_ref[...] + b1_ref[...])
  h2_ref[...] = x2 @ w2_ref[...]
  wa = w2_ref[...] @ jnp.concatenate([as_ref[...], ad_ref[...]], axis=1)
  al = x2 @ wa
  als_ref[...] = al[:, 0:1]
  ald_ref[...] = al[:, 1:2]


_tc_mid = pl.pallas_call(
    _tc_mid_body,
    out_shape=(
        jax.ShapeDtypeStruct((N, OUT_CH), jnp.float32),
        jax.ShapeDtypeStruct((N, 1), jnp.float32),
        jax.ShapeDtypeStruct((N, 1), jnp.float32),
    ),
)


def _tc_post_body(agg_ref, den_ref, b2_ref, out_ref):
  agg = agg_ref[0] + agg_ref[1]
  den = jnp.maximum(den_ref[0] + den_ref[1], 1e-16)
  out = agg[:, :N].T * (1.0 / den[:N])[:, None] + b2_ref[...]
  out_ref[...] = jax.nn.relu(out)


_tc_post = pl.pallas_call(
    _tc_post_body,
    out_shape=jax.ShapeDtypeStruct((N, OUT_CH), jnp.float32),
)


@jax.jit
def kernel(x, edge_index, W1, a_src1, a_dst1, b1, W2, a_src2, a_dst2, b2):
  src = edge_index[0].astype(jnp.int32)
  dst = edge_index[1].astype(jnp.int32)
  src3a = jnp.pad(src.reshape(NW, EPT), ((0, 0), (0, NCH1 * G1 - EPT))
                  ).reshape(NW, NCH1, G1)
  dst3a = jnp.pad(dst.reshape(NW, EPT), ((0, 0), (0, NCH1 * G1 - EPT))
                  ).reshape(NW, NCH1, G1)
  src3 = jnp.pad(src.reshape(NW, EPT), ((0, 0), (0, NCH * G - EPT))
                 ).reshape(NW, NCH, G)
  dst3 = jnp.pad(dst.reshape(NW, EPT), ((0, 0), (0, NCH * G - EPT))
                 ).reshape(NW, NCH, G)

  als1, ald1 = _tc_pre1(x, W1, a_src1.reshape(HID, 1),
                        a_dst1.reshape(HID, 1))
  als1p = jnp.pad(als1.reshape(N), (0, NA - N))
  ald1p = jnp.pad(ald1.reshape(N), (0, NA - N))
  agg1, den1 = _edge_k1(x, src3a, dst3a, als1p, ald1p)
  h2, als2, ald2 = _tc_mid(agg1, den1, W1, b1.reshape(1, HID), W2,
                           a_src2.reshape(OUT_CH, 1),
                           a_dst2.reshape(OUT_CH, 1))
  h2f = jnp.pad(h2.T, ((0, 0), (0, NP - N))).reshape(OUT_CH * NP)
  als2p = jnp.pad(als2.reshape(N), (0, NA - N))
  ald2p = jnp.pad(ald2.reshape(N), (0, NA - N))
  agg2, den2 = _edge_k2(h2f, src3, dst3, als2p, ald2p)
  return _tc_post(agg2, den2, b2.reshape(1, OUT_CH))
